# baseline (speedup 1.0000x reference)
import functools

import jax
import jax.numpy as jnp
from jax import lax
from jax.experimental import pallas as pl
from jax.experimental.pallas import tpu as pltpu
from jax.experimental.pallas import tpu_sc as plsc


def kernel(values_dyn, values_static, dyn_table, static_table):
    (b_dyn,) = values_dyn.shape
    (b_st,) = values_static.shape
    dim = dyn_table.shape[1]

    info = plsc.get_sparse_core_info()
    nw = info.num_cores * info.num_subcores
    nc = info.num_cores

    bpw_dyn = b_dyn // nw
    bpw_st = b_st // nw
    ch = 512
    n_ch = bpw_dyn // ch

    assert b_dyn % (nw * ch) == 0 and b_st % nw == 0 and bpw_st <= ch

    mesh = plsc.VectorSubcoreMesh(core_axis_name="c", subcore_axis_name="s")

    @functools.partial(
        pl.kernel,
        out_type=(
            jax.ShapeDtypeStruct((b_dyn, dim), jnp.float32),
            jax.ShapeDtypeStruct((b_st, dim), jnp.float32),
        ),
        mesh=mesh,
        scratch_types=[
            pltpu.VMEM((bpw_dyn,), jnp.int32),
            pltpu.VMEM((ch, dim), jnp.float32),
            pltpu.VMEM((bpw_st,), jnp.int32),
            pltpu.SemaphoreType.DMA,
        ],
        compiler_params=pltpu.CompilerParams(use_tc_tiling_on_sc=False),
    )
    def emb_kernel(vdyn_hbm, vst_hbm, dtab_hbm, stab_hbm, out_dyn, out_st,
                   idx_v, rows_v, sidx_v, sem):
        wid = lax.axis_index("s") * nc + lax.axis_index("c")

        st_base = wid * bpw_st
        pltpu.sync_copy(vst_hbm.at[pl.ds(st_base, bpw_st)], sidx_v)
        pltpu.async_copy(stab_hbm.at[sidx_v], rows_v.at[pl.ds(0, bpw_st)], sem).wait()
        pltpu.sync_copy(rows_v.at[pl.ds(0, bpw_st)], out_st.at[pl.ds(st_base, bpw_st)])

        base = wid * bpw_dyn
        pltpu.sync_copy(vdyn_hbm.at[pl.ds(base, bpw_dyn)], idx_v)

        def body(g, carry):
            pltpu.async_copy(
                dtab_hbm.at[idx_v.at[pl.ds(g * ch, ch)]], rows_v, sem).wait()
            pltpu.sync_copy(rows_v, out_dyn.at[pl.ds(base + g * ch, ch)])
            return carry

        lax.fori_loop(0, n_ch, body, 0)

    return emb_kernel(values_dyn, values_static, dyn_table, static_table)


# speedup vs baseline: 1.0230x; 1.0230x over previous
import functools

import jax
import jax.numpy as jnp
from jax import lax
from jax.experimental import pallas as pl
from jax.experimental.pallas import tpu as pltpu
from jax.experimental.pallas import tpu_sc as plsc


def kernel(values_dyn, values_static, dyn_table, static_table):
    (b_dyn,) = values_dyn.shape
    (b_st,) = values_static.shape
    dim = dyn_table.shape[1]

    info = plsc.get_sparse_core_info()
    nw = info.num_cores * info.num_subcores
    nc = info.num_cores

    bpw_dyn = b_dyn // nw
    bpw_st = b_st // nw
    ch = 512
    n_ch = bpw_dyn // ch

    assert b_dyn % (nw * ch) == 0 and b_st % nw == 0 and n_ch % 2 == 0

    mesh = plsc.VectorSubcoreMesh(core_axis_name="c", subcore_axis_name="s")

    @functools.partial(
        pl.kernel,
        out_type=(
            jax.ShapeDtypeStruct((b_dyn, dim), jnp.float32),
            jax.ShapeDtypeStruct((b_st, dim), jnp.float32),
        ),
        mesh=mesh,
        scratch_types=[
            pltpu.VMEM((bpw_dyn,), jnp.int32),
            pltpu.VMEM((2, ch, dim), jnp.float32),
            pltpu.VMEM((bpw_st,), jnp.int32),
            pltpu.VMEM((bpw_st, dim), jnp.float32),
            pltpu.SemaphoreType.DMA,
            pltpu.SemaphoreType.DMA,
            pltpu.SemaphoreType.DMA,
            pltpu.SemaphoreType.DMA,
            pltpu.SemaphoreType.DMA,
        ],
        compiler_params=pltpu.CompilerParams(use_tc_tiling_on_sc=False),
    )
    def emb_kernel(vdyn_hbm, vst_hbm, dtab_hbm, stab_hbm, out_dyn, out_st,
                   idx_v, rows_v, sidx_v, srows_v,
                   sem_g0, sem_g1, sem_o0, sem_o1, sem_s):
        wid = lax.axis_index("s") * nc + lax.axis_index("c")
        base = wid * bpw_dyn
        st_base = wid * bpw_st
        sem_g = (sem_g0, sem_g1)
        sem_o = (sem_o0, sem_o1)

        def gth(g, b):
            return pltpu.make_async_copy(
                dtab_hbm.at[idx_v.at[pl.ds(g * ch, ch)]], rows_v.at[b], sem_g[b])

        def outc(g, b):
            return pltpu.make_async_copy(
                rows_v.at[b], out_dyn.at[pl.ds(base + g * ch, ch)], sem_o[b])

        pltpu.sync_copy(vst_hbm.at[pl.ds(st_base, bpw_st)], sidx_v)
        pltpu.sync_copy(vdyn_hbm.at[pl.ds(base, bpw_dyn)], idx_v)
        pltpu.async_copy(stab_hbm.at[sidx_v], srows_v, sem_s)

        gth(0, 0).start()

        def step(g, b):
            gth(g, b).wait()
            outc(g - 1, 1 - b).wait()
            gth(g + 1, 1 - b).start()
            outc(g, b).start()

        def body(p, carry):
            step(2 * p + 1, 1)
            step(2 * p + 2, 0)
            return carry

        gth(0, 0).wait()
        gth(1, 1).start()
        outc(0, 0).start()
        lax.fori_loop(0, (n_ch - 2) // 2, body, 0)
        g_last = n_ch - 1
        gth(g_last, 1).wait()
        outc(g_last - 1, 0).wait()
        outc(g_last, 1).start()
        outc(g_last, 1).wait()
        pltpu.make_async_copy(stab_hbm.at[sidx_v], srows_v, sem_s).wait()
        pltpu.sync_copy(srows_v, out_st.at[pl.ds(st_base, bpw_st)])

    return emb_kernel(values_dyn, values_static, dyn_table, static_table)


# speedup vs baseline: 1.4347x; 1.4024x over previous
import functools

import jax
import jax.numpy as jnp
from jax import lax
from jax.experimental import pallas as pl
from jax.experimental.pallas import tpu as pltpu
from jax.experimental.pallas import tpu_sc as plsc

LANES = 16
PAD = 128


def kernel(values_dyn, values_static, dyn_table, static_table):
    (b_dyn,) = values_dyn.shape
    (b_st,) = values_static.shape
    v_dyn, dim = dyn_table.shape
    v_st = static_table.shape[0]

    t2 = jnp.pad(dyn_table, ((0, 0), (0, PAD - dim))).reshape(2 * v_dyn, dim)
    s2 = jnp.pad(static_table, ((0, 0), (0, PAD - dim))).reshape(2 * v_st, dim)

    info = plsc.get_sparse_core_info()
    nw = info.num_cores * info.num_subcores
    nc = info.num_cores

    bpw_dyn = b_dyn // nw
    bpw_st = b_st // nw
    ch = 512
    n_ch = bpw_dyn // ch
    n_st = bpw_st // ch

    assert b_dyn % (nw * ch) == 0 and b_st % (nw * ch) == 0
    assert n_ch % 2 == 0 and dim == 64

    mesh = plsc.VectorSubcoreMesh(core_axis_name="c", subcore_axis_name="s")

    @functools.partial(
        pl.kernel,
        out_type=(
            jax.ShapeDtypeStruct((b_dyn, PAD), jnp.float32),
            jax.ShapeDtypeStruct((b_st, PAD), jnp.float32),
        ),
        mesh=mesh,
        scratch_types=[
            pltpu.VMEM((bpw_dyn,), jnp.int32),
            pltpu.VMEM((bpw_st,), jnp.int32),
            pltpu.VMEM((ch,), jnp.int32),
            pltpu.VMEM((ch,), jnp.int32),
            pltpu.VMEM((ch, 64), jnp.float32),
            pltpu.VMEM((ch, 64), jnp.float32),
            pltpu.SemaphoreType.DMA,
            pltpu.SemaphoreType.DMA,
            pltpu.SemaphoreType.DMA,
            pltpu.SemaphoreType.DMA,
        ],
        compiler_params=pltpu.CompilerParams(
            use_tc_tiling_on_sc=False, needs_layout_passes=False),
    )
    def emb_kernel(vdyn_hbm, vst_hbm, t2_hbm, s2_hbm, out_dyn, out_st,
                   idx_v, sidx_v, pid0, pid1, gbuf0, gbuf1,
                   sem_g0, sem_g1, sem_o0, sem_o1):
        wid = lax.axis_index("s") * nc + lax.axis_index("c")
        pid = (pid0, pid1)
        gbuf = (gbuf0, gbuf1)
        sem_g = (sem_g0, sem_g1)
        sem_o = (sem_o0, sem_o1)
        base = wid * bpw_dyn
        st_base = wid * bpw_st

        pltpu.sync_copy(vst_hbm.at[pl.ds(st_base, bpw_st)], sidx_v)
        pltpu.sync_copy(vdyn_hbm.at[pl.ds(base, bpw_dyn)], idx_v)

        def prep(iref, g, b):
            def pp(q, carry):
                blk = iref[pl.ds(g * ch + q * LANES, LANES)]
                pid[b][pl.ds(q * LANES, LANES)] = lax.shift_left(blk, 1)
                return carry
            lax.fori_loop(0, ch // LANES, pp, 0)

        def gth(tref, g, b):
            return pltpu.make_async_copy(tref.at[pid[b]], gbuf[b], sem_g[b])

        def outc(oref, gbase, g, b):
            dst = oref.at[pl.ds(gbase + g * ch, ch), pl.ds(0, dim)]
            return pltpu.make_async_copy(gbuf[b], dst, sem_o[b])

        prep(idx_v, 0, 0)
        gth(t2_hbm, 0, 0).start()
        prep(idx_v, 1, 1)
        gth(t2_hbm, 0, 0).wait()
        gth(t2_hbm, 1, 1).start()
        outc(out_dyn, base, 0, 0).start()

        def step(g, b):
            gth(t2_hbm, g, b).wait()
            prep(idx_v, g + 1, 1 - b)
            outc(out_dyn, base, g - 1, 1 - b).wait()
            gth(t2_hbm, g + 1, 1 - b).start()
            outc(out_dyn, base, g, b).start()

        def body(p, carry):
            step(2 * p + 1, 1)
            step(2 * p + 2, 0)
            return carry

        lax.fori_loop(0, (n_ch - 2) // 2, body, 0)
        g_last = n_ch - 1
        gth(t2_hbm, g_last, 1).wait()
        outc(out_dyn, base, g_last - 1, 0).wait()
        outc(out_dyn, base, g_last, 1).start()
        outc(out_dyn, base, g_last, 1).wait()

        for g in range(n_st):
            b = g % 2
            prep(sidx_v, g, b)
            gth(s2_hbm, g, b).start()
            gth(s2_hbm, g, b).wait()
            outc(out_st, st_base, g, b).start()
            outc(out_st, st_base, g, b).wait()

    out_dyn, out_st = emb_kernel(values_dyn, values_static, t2, s2)
    return (out_dyn[:, :dim], out_st[:, :dim])


# speedup vs baseline: 1.4364x; 1.0012x over previous
import functools

import jax
import jax.numpy as jnp
from jax import lax
from jax.experimental import pallas as pl
from jax.experimental.pallas import tpu as pltpu
from jax.experimental.pallas import tpu_sc as plsc

LANES = 16
PAD = 128


def kernel(values_dyn, values_static, dyn_table, static_table):
    (b_dyn,) = values_dyn.shape
    (b_st,) = values_static.shape
    v_dyn, dim = dyn_table.shape
    v_st = static_table.shape[0]

    t2 = jnp.pad(dyn_table, ((0, 0), (0, PAD - dim))).reshape(2 * v_dyn, dim)
    s2 = jnp.pad(static_table, ((0, 0), (0, PAD - dim))).reshape(2 * v_st, dim)

    info = plsc.get_sparse_core_info()
    nw = info.num_cores * info.num_subcores
    nc = info.num_cores

    bpw_dyn = b_dyn // nw
    bpw_st = b_st // nw
    ch = 640
    n_ch = bpw_dyn // ch

    assert b_dyn % (nw * ch) == 0 and bpw_st % LANES == 0 and bpw_st <= ch
    assert n_ch % 2 == 0 and dim == 64

    mesh = plsc.VectorSubcoreMesh(core_axis_name="c", subcore_axis_name="s")

    @functools.partial(
        pl.kernel,
        out_type=(
            jax.ShapeDtypeStruct((b_dyn, PAD), jnp.float32),
            jax.ShapeDtypeStruct((b_st, PAD), jnp.float32),
        ),
        mesh=mesh,
        scratch_types=[
            pltpu.VMEM((bpw_dyn,), jnp.int32),
            pltpu.VMEM((bpw_st,), jnp.int32),
            pltpu.VMEM((ch,), jnp.int32),
            pltpu.VMEM((ch,), jnp.int32),
            pltpu.VMEM((ch, 64), jnp.float32),
            pltpu.VMEM((ch, 64), jnp.float32),
            pltpu.SemaphoreType.DMA,
            pltpu.SemaphoreType.DMA,
            pltpu.SemaphoreType.DMA,
            pltpu.SemaphoreType.DMA,
        ],
        compiler_params=pltpu.CompilerParams(
            use_tc_tiling_on_sc=False, needs_layout_passes=False),
    )
    def emb_kernel(vdyn_hbm, vst_hbm, t2_hbm, s2_hbm, out_dyn, out_st,
                   idx_v, sidx_v, pid0, pid1, gbuf0, gbuf1,
                   sem_g0, sem_g1, sem_o0, sem_o1):
        wid = lax.axis_index("s") * nc + lax.axis_index("c")
        pid = (pid0, pid1)
        gbuf = (gbuf0, gbuf1)
        sem_g = (sem_g0, sem_g1)
        sem_o = (sem_o0, sem_o1)
        base = wid * bpw_dyn
        st_base = wid * bpw_st

        pltpu.sync_copy(vst_hbm.at[pl.ds(st_base, bpw_st)], sidx_v)
        pltpu.sync_copy(vdyn_hbm.at[pl.ds(base, bpw_dyn)], idx_v)

        def prep(iref, g, b):
            def pp(q, carry):
                blk = iref[pl.ds(g * ch + q * LANES, LANES)]
                pid[b][pl.ds(q * LANES, LANES)] = lax.shift_left(blk, 1)
                return carry
            lax.fori_loop(0, ch // LANES, pp, 0)

        def gth(tref, g, b):
            return pltpu.make_async_copy(tref.at[pid[b]], gbuf[b], sem_g[b])

        def outc(oref, gbase, g, b):
            dst = oref.at[pl.ds(gbase + g * ch, ch), pl.ds(0, dim)]
            return pltpu.make_async_copy(gbuf[b], dst, sem_o[b])

        prep(idx_v, 0, 0)
        gth(t2_hbm, 0, 0).start()
        prep(idx_v, 1, 1)
        gth(t2_hbm, 0, 0).wait()
        gth(t2_hbm, 1, 1).start()
        outc(out_dyn, base, 0, 0).start()

        def step(g, b):
            gth(t2_hbm, g, b).wait()
            prep(idx_v, g + 1, 1 - b)
            outc(out_dyn, base, g - 1, 1 - b).wait()
            gth(t2_hbm, g + 1, 1 - b).start()
            outc(out_dyn, base, g, b).start()

        def body(p, carry):
            step(2 * p + 1, 1)
            step(2 * p + 2, 0)
            return carry

        lax.fori_loop(0, (n_ch - 2) // 2, body, 0)
        g_last = n_ch - 1
        gth(t2_hbm, g_last, 1).wait()
        outc(out_dyn, base, g_last - 1, 0).wait()
        outc(out_dyn, base, g_last, 1).start()
        outc(out_dyn, base, g_last, 1).wait()

        def pps(q, carry):
            blk = sidx_v[pl.ds(q * LANES, LANES)]
            pid0[pl.ds(q * LANES, LANES)] = lax.shift_left(blk, 1)
            return carry
        lax.fori_loop(0, bpw_st // LANES, pps, 0)
        scp = pltpu.make_async_copy(
            s2_hbm.at[pid0.at[pl.ds(0, bpw_st)]],
            gbuf0.at[pl.ds(0, bpw_st)], sem_g0)
        scp.start()
        scp.wait()
        ocp = pltpu.make_async_copy(
            gbuf0.at[pl.ds(0, bpw_st)],
            out_st.at[pl.ds(st_base, bpw_st), pl.ds(0, dim)], sem_o0)
        ocp.start()
        ocp.wait()

    out_dyn, out_st = emb_kernel(values_dyn, values_static, t2, s2)
    return (out_dyn[:, :dim], out_st[:, :dim])
